# baseline (speedup 1.0000x reference)
import functools

import jax
import jax.numpy as jnp
from jax import lax
from jax.experimental import pallas as pl
from jax.experimental.pallas import tpu as pltpu
from jax.experimental.pallas import tpu_sc as plsc

_N_LEVELS = 16
_NFEAT = 2
_LOG2 = 19
_HASHMAP = 2 ** _LOG2
_MASK = _HASHMAP - 1
_BASE, _FINEST = 16, 512
_P2 = 2654435761 & _MASK
_P3 = 805459861 & _MASK
_BLKMASK = _MASK & ~127
_N_POINTS = 262144
_NOUT = _N_LEVELS * _NFEAT

_NC, _NS, _L = 2, 16, 16
_NW = _NC * _NS
_NPT = _N_POINTS // _NW
_C = 1024
_NCHUNK = _NPT // _C
_G16 = _C // _L


def _resolutions():
    growth = (_FINEST / _BASE) ** (1.0 / (_N_LEVELS - 1))
    return [int(_BASE * growth ** i) for i in range(_N_LEVELS)]


_RCHUNK = 8192


def _delinearize(flat):
    u = _RCHUNK // 128

    def body(i_ref, o_ref):
        for j in range(u):
            o_ref[:, pl.ds(j * 128, 128)] = i_ref[:, j, :]

    flat3 = flat.reshape(_NOUT, _N_POINTS // 128, 128)
    out = pl.pallas_call(
        body,
        grid=(_N_POINTS // _RCHUNK,),
        in_specs=[pl.BlockSpec((_NOUT, u, 128), lambda i: (0, i, 0))],
        out_specs=pl.BlockSpec((_NOUT, _RCHUNK), lambda i: (0, i)),
        out_shape=jax.ShapeDtypeStruct((_NOUT, _N_POINTS), jnp.float32),
    )(flat3)
    return out.T


def kernel(x, tables):
    tables_flat = (
        tables.transpose(0, 2, 1)
        .reshape(_N_LEVELS, _NFEAT, _HASHMAP // 128, 128)
        .transpose(0, 2, 1, 3)
        .reshape(-1)
    )
    x_flat = x.reshape(-1)
    res = _resolutions()
    mesh = plsc.VectorSubcoreMesh(core_axis_name="c", subcore_axis_name="s")

    @functools.partial(
        pl.kernel,
        out_type=jax.ShapeDtypeStruct((_N_POINTS * _NOUT,), jnp.float32),
        mesh=mesh,
        compiler_params=pltpu.CompilerParams(needs_layout_passes=False),
        scratch_types=[
            pltpu.VMEM((3 * _C,), jnp.float32),
            pltpu.VMEM((16 * _C,), jnp.int32),
            pltpu.VMEM((16 * _C,), jnp.int32),
            pltpu.VMEM((16 * _C,), jnp.float32),
            pltpu.VMEM((16 * _C,), jnp.float32),
            pltpu.VMEM((3 * _C,), jnp.float32),
            pltpu.VMEM((3 * _C,), jnp.float32),
            pltpu.VMEM((_C * _NOUT,), jnp.float32),
            pltpu.SemaphoreType.DMA,
            pltpu.SemaphoreType.DMA,
            pltpu.SemaphoreType.DMA,
        ],
    )
    def _k(x_hbm, tab_hbm, out_hbm, xbuf, idxb0, idxb1, featb0, featb1,
           fracb0, fracb1, outbuf, gsem0, gsem1, osem):
        wid = lax.axis_index("s") * _NC + lax.axis_index("c")
        base_pt = wid * _NPT
        idxbufs = (idxb0, idxb1)
        featbufs = (featb0, featb1)
        fracbufs = (fracb0, fracb1)
        gsems = (gsem0, gsem1)
        lanes = lax.iota(jnp.int32, _L)
        lanes3 = lanes * 3

        def make_pass_a(lvl, idxb, fracb):
            rf = float(res[lvl])
            lvl_base = lvl << (_LOG2 + 1)

            def pass_a(g, carry):
                rows3 = (g * _L) * 3 + lanes3
                xv = plsc.load_gather(xbuf, [rows3])
                yv = plsc.load_gather(xbuf, [rows3 + 1])
                zv = plsc.load_gather(xbuf, [rows3 + 2])
                sx = xv * rf
                sy = yv * rf
                sz = zv * rf
                fxi = sx.astype(jnp.int32)
                fyi = sy.astype(jnp.int32)
                fzi = sz.astype(jnp.int32)
                q = g * _L
                fracb[pl.ds(q, _L)] = sx - fxi.astype(jnp.float32)
                fracb[pl.ds(_C + q, _L)] = sy - fyi.astype(jnp.float32)
                fracb[pl.ds(2 * _C + q, _L)] = sz - fzi.astype(jnp.float32)
                hx0 = fxi
                hx1 = fxi + 1
                hy0 = fyi * _P2
                hy1 = (fyi + 1) * _P2
                hz0 = fzi * _P3
                hz1 = (fzi + 1) * _P3
                corners = ((hx0, hy0, hz0), (hx1, hy0, hz0),
                           (hx0, hy1, hz0), (hx1, hy1, hz0),
                           (hx0, hy0, hz1), (hx1, hy0, hz1),
                           (hx0, hy1, hz1), (hx1, hy1, hz1))
                for c, (hx, hy, hz) in enumerate(corners):
                    h = (hx ^ hy ^ hz) & _MASK
                    e0 = lvl_base + h + (h & _BLKMASK)
                    idxb[pl.ds(c * _C + q, _L)] = e0
                    idxb[pl.ds(8 * _C + c * _C + q, _L)] = e0 + 128
                return carry

            return pass_a

        def make_pass_b(lvl, featb, fracb):
            def pass_b(g, carry):
                q = g * _L
                wx = fracb[pl.ds(q, _L)]
                wy = fracb[pl.ds(_C + q, _L)]
                wz = fracb[pl.ds(2 * _C + q, _L)]
                ux = 1.0 - wx
                uy = 1.0 - wy
                uz = 1.0 - wz
                pa = ux * uy
                pb = wx * uy
                pc = ux * wy
                pd = wx * wy
                ws = (pa * uz, pb * uz, pc * uz, pd * uz,
                      pa * wz, pb * wz, pc * wz, pd * wz)
                acc0 = ws[0] * featb[pl.ds(q, _L)]
                acc1 = ws[0] * featb[pl.ds(8 * _C + q, _L)]
                for c in range(1, 8):
                    acc0 = acc0 + ws[c] * featb[pl.ds(c * _C + q, _L)]
                    acc1 = acc1 + ws[c] * featb[pl.ds(8 * _C + c * _C + q, _L)]
                outbuf[pl.ds(2 * lvl * _C + q, _L)] = acc0
                outbuf[pl.ds((2 * lvl + 1) * _C + q, _L)] = acc1
                return carry

            return pass_b

        def chunk_body(chunk, carry0):
            c0 = chunk * _C
            pltpu.sync_copy(x_hbm.at[pl.ds((base_pt + c0) * 3, _C * 3)], xbuf)

            ghandles = [None, None]
            ohandles = []

            def emit_out(lvl):
                for t in (0, 1):
                    q = 2 * lvl + t
                    ohandles.append(pltpu.async_copy(
                        outbuf.at[pl.ds(q * _C, _C)],
                        out_hbm.at[pl.ds(q * _N_POINTS + base_pt + c0, _C)],
                        osem,
                    ))

            lax.fori_loop(0, _G16, make_pass_a(0, idxbufs[0], fracbufs[0]), 0)
            ghandles[0] = pltpu.async_copy(
                tab_hbm.at[idxbufs[0]], featbufs[0], gsems[0])
            for lvl in range(1, _N_LEVELS):
                b = lvl & 1
                pb_ = b ^ 1
                lax.fori_loop(0, _G16, make_pass_a(lvl, idxbufs[b], fracbufs[b]), 0)
                ghandles[b] = pltpu.async_copy(
                    tab_hbm.at[idxbufs[b]], featbufs[b], gsems[b])
                ghandles[pb_].wait()
                lax.fori_loop(0, _G16, make_pass_b(lvl - 1, featbufs[pb_], fracbufs[pb_]), 0)
                emit_out(lvl - 1)
            last = (_N_LEVELS - 1) & 1
            ghandles[last].wait()
            lax.fori_loop(0, _G16, make_pass_b(_N_LEVELS - 1, featbufs[last], fracbufs[last]), 0)
            emit_out(_N_LEVELS - 1)
            for h in ohandles:
                h.wait()
            return carry0

        lax.fori_loop(0, _NCHUNK, chunk_body, 0)

    return _delinearize(_k(x_flat, tables_flat))


# speedup vs baseline: 1.0156x; 1.0156x over previous
import functools

import jax
import jax.numpy as jnp
from jax import lax
from jax.experimental import pallas as pl
from jax.experimental.pallas import tpu as pltpu
from jax.experimental.pallas import tpu_sc as plsc

_N_LEVELS = 16
_NFEAT = 2
_LOG2 = 19
_HASHMAP = 2 ** _LOG2
_MASK = _HASHMAP - 1
_BASE, _FINEST = 16, 512
_P2 = 2654435761 & _MASK
_P3 = 805459861 & _MASK
_BLKMASK = _MASK & ~127
_N_POINTS = 262144
_NOUT = _N_LEVELS * _NFEAT

_NC, _NS, _L = 2, 16, 16
_NW = _NC * _NS
_NPT = _N_POINTS // _NW
_C = 1024
_NCHUNK = _NPT // _C
_G16 = _C // _L


def _resolutions():
    growth = (_FINEST / _BASE) ** (1.0 / (_N_LEVELS - 1))
    return [int(_BASE * growth ** i) for i in range(_N_LEVELS)]


_RCHUNK = 8192


def _delinearize(flat):
    u = _RCHUNK // 128

    def body(i_ref, o_ref):
        for j in range(u):
            o_ref[:, pl.ds(j * 128, 128)] = i_ref[:, j, :]

    flat3 = flat.reshape(_NOUT, _N_POINTS // 128, 128)
    out = pl.pallas_call(
        body,
        grid=(_N_POINTS // _RCHUNK,),
        in_specs=[pl.BlockSpec((_NOUT, u, 128), lambda i: (0, i, 0))],
        out_specs=pl.BlockSpec((_NOUT, _RCHUNK), lambda i: (0, i)),
        out_shape=jax.ShapeDtypeStruct((_NOUT, _N_POINTS), jnp.float32),
    )(flat3)
    return out.T


def kernel(x, tables):
    tables_flat = (
        tables.transpose(0, 2, 1)
        .reshape(_N_LEVELS, _NFEAT, _HASHMAP // 128, 128)
        .transpose(0, 2, 1, 3)
        .reshape(-1)
    )
    x_flat = x.reshape(-1)
    res = _resolutions()
    mesh = plsc.VectorSubcoreMesh(core_axis_name="c", subcore_axis_name="s")

    @functools.partial(
        pl.kernel,
        out_type=jax.ShapeDtypeStruct((_N_POINTS * _NOUT,), jnp.float32),
        mesh=mesh,
        compiler_params=pltpu.CompilerParams(needs_layout_passes=False),
        scratch_types=[
            pltpu.VMEM((3 * _C,), jnp.float32),
            pltpu.VMEM((16 * _C,), jnp.int32),
            pltpu.VMEM((16 * _C,), jnp.int32),
            pltpu.VMEM((16 * _C,), jnp.float32),
            pltpu.VMEM((16 * _C,), jnp.float32),
            pltpu.VMEM((3 * _C,), jnp.float32),
            pltpu.VMEM((3 * _C,), jnp.float32),
            pltpu.VMEM((_C * _NOUT,), jnp.float32),
            pltpu.SemaphoreType.DMA,
            pltpu.SemaphoreType.DMA,
            pltpu.SemaphoreType.DMA,
            pltpu.SemaphoreType.DMA,
            pltpu.SemaphoreType.DMA,
        ],
    )
    def _k(x_hbm, tab_hbm, out_hbm, xbuf, idxb0, idxb1, featb0, featb1,
           fracb0, fracb1, outbuf, gsem0a, gsem0b, gsem1a, gsem1b, osem):
        wid = lax.axis_index("s") * _NC + lax.axis_index("c")
        base_pt = wid * _NPT
        idxbufs = (idxb0, idxb1)
        featbufs = (featb0, featb1)
        fracbufs = (fracb0, fracb1)
        gsems = ((gsem0a, gsem0b), (gsem1a, gsem1b))
        lanes = lax.iota(jnp.int32, _L)
        lanes3 = lanes * 3

        def make_pass_a(lvl, idxb, fracb):
            rf = float(res[lvl])
            lvl_base = lvl << (_LOG2 + 1)

            def pass_a(g, carry):
                rows3 = (g * _L) * 3 + lanes3
                xv = plsc.load_gather(xbuf, [rows3])
                yv = plsc.load_gather(xbuf, [rows3 + 1])
                zv = plsc.load_gather(xbuf, [rows3 + 2])
                sx = xv * rf
                sy = yv * rf
                sz = zv * rf
                fxi = sx.astype(jnp.int32)
                fyi = sy.astype(jnp.int32)
                fzi = sz.astype(jnp.int32)
                q = g * _L
                fracb[pl.ds(q, _L)] = sx - fxi.astype(jnp.float32)
                fracb[pl.ds(_C + q, _L)] = sy - fyi.astype(jnp.float32)
                fracb[pl.ds(2 * _C + q, _L)] = sz - fzi.astype(jnp.float32)
                hx0 = fxi
                hx1 = fxi + 1
                hy0 = fyi * _P2
                hy1 = (fyi + 1) * _P2
                hz0 = fzi * _P3
                hz1 = (fzi + 1) * _P3
                corners = ((hx0, hy0, hz0), (hx1, hy0, hz0),
                           (hx0, hy1, hz0), (hx1, hy1, hz0),
                           (hx0, hy0, hz1), (hx1, hy0, hz1),
                           (hx0, hy1, hz1), (hx1, hy1, hz1))
                for c, (hx, hy, hz) in enumerate(corners):
                    h = (hx ^ hy ^ hz) & _MASK
                    e0 = lvl_base + h + (h & _BLKMASK)
                    idxb[pl.ds(c * _C + q, _L)] = e0
                    idxb[pl.ds(8 * _C + c * _C + q, _L)] = e0 + 128
                return carry

            return pass_a

        def make_pass_b(lvl, featb, fracb):
            def pass_b(g, carry):
                q = g * _L
                wx = fracb[pl.ds(q, _L)]
                wy = fracb[pl.ds(_C + q, _L)]
                wz = fracb[pl.ds(2 * _C + q, _L)]
                ux = 1.0 - wx
                uy = 1.0 - wy
                uz = 1.0 - wz
                pa = ux * uy
                pb = wx * uy
                pc = ux * wy
                pd = wx * wy
                ws = (pa * uz, pb * uz, pc * uz, pd * uz,
                      pa * wz, pb * wz, pc * wz, pd * wz)
                acc0 = ws[0] * featb[pl.ds(q, _L)]
                acc1 = ws[0] * featb[pl.ds(8 * _C + q, _L)]
                for c in range(1, 8):
                    acc0 = acc0 + ws[c] * featb[pl.ds(c * _C + q, _L)]
                    acc1 = acc1 + ws[c] * featb[pl.ds(8 * _C + c * _C + q, _L)]
                outbuf[pl.ds(2 * lvl * _C + q, _L)] = acc0
                outbuf[pl.ds((2 * lvl + 1) * _C + q, _L)] = acc1
                return carry

            return pass_b

        def chunk_body(chunk, carry0):
            c0 = chunk * _C
            pltpu.sync_copy(x_hbm.at[pl.ds((base_pt + c0) * 3, _C * 3)], xbuf)

            ghandles = [None, None]
            ohandles = []

            def emit_out(lvl):
                for t in (0, 1):
                    q = 2 * lvl + t
                    ohandles.append(pltpu.async_copy(
                        outbuf.at[pl.ds(q * _C, _C)],
                        out_hbm.at[pl.ds(q * _N_POINTS + base_pt + c0, _C)],
                        osem,
                    ))

            def start_gather(b):
                idxb, featb, (sa, sb) = idxbufs[b], featbufs[b], gsems[b]
                return (
                    pltpu.async_copy(tab_hbm.at[idxb.at[pl.ds(0, 8 * _C)]],
                                     featb.at[pl.ds(0, 8 * _C)], sa),
                    pltpu.async_copy(
                        tab_hbm.at[idxb.at[pl.ds(8 * _C, 8 * _C)]],
                        featb.at[pl.ds(8 * _C, 8 * _C)], sb),
                )

            lax.fori_loop(0, _G16, make_pass_a(0, idxbufs[0], fracbufs[0]), 0)
            ghandles[0] = start_gather(0)
            for lvl in range(1, _N_LEVELS):
                b = lvl & 1
                pb_ = b ^ 1
                lax.fori_loop(0, _G16, make_pass_a(lvl, idxbufs[b], fracbufs[b]), 0)
                ghandles[b] = start_gather(b)
                for h in ghandles[pb_]:
                    h.wait()
                lax.fori_loop(0, _G16, make_pass_b(lvl - 1, featbufs[pb_], fracbufs[pb_]), 0)
                emit_out(lvl - 1)
            last = (_N_LEVELS - 1) & 1
            for h in ghandles[last]:
                h.wait()
            lax.fori_loop(0, _G16, make_pass_b(_N_LEVELS - 1, featbufs[last], fracbufs[last]), 0)
            emit_out(_N_LEVELS - 1)
            for h in ohandles:
                h.wait()
            return carry0

        lax.fori_loop(0, _NCHUNK, chunk_body, 0)

    return _delinearize(_k(x_flat, tables_flat))


# speedup vs baseline: 1.2149x; 1.1963x over previous
import functools

import jax
import jax.numpy as jnp
from jax import lax
from jax.experimental import pallas as pl
from jax.experimental.pallas import tpu as pltpu
from jax.experimental.pallas import tpu_sc as plsc

_N_LEVELS = 16
_NFEAT = 2
_LOG2 = 19
_HASHMAP = 2 ** _LOG2
_MASK = _HASHMAP - 1
_BASE, _FINEST = 16, 512
_P2 = 2654435761 & _MASK
_P3 = 805459861 & _MASK
_BLKMASK = _MASK & ~127
_N_POINTS = 262144
_NOUT = _N_LEVELS * _NFEAT

_NC, _NS, _L = 2, 16, 16
_NW = _NC * _NS
_NPT = _N_POINTS // _NW
_C = 512
_NCHUNK = _NPT // _C
_G16 = _C // _L


def _resolutions():
    growth = (_FINEST / _BASE) ** (1.0 / (_N_LEVELS - 1))
    return [int(_BASE * growth ** i) for i in range(_N_LEVELS)]


_DENSE_L = 3
_DENSE_R = [r + 1 for r in _resolutions()[:_DENSE_L]]
_DENSE_PLANE = [(_r ** 3 + 7) & ~7 for _r in _DENSE_R]
_DENSE_OFF = []
_dtot = 0
for _p in _DENSE_PLANE:
    _DENSE_OFF.append(_dtot)
    _dtot += 2 * _p
_DTOT = _dtot


_RCHUNK = 8192


def _delinearize(flat):
    u = _RCHUNK // 128

    def body(i_ref, o_ref):
        for j in range(u):
            o_ref[:, pl.ds(j * 128, 128)] = i_ref[:, j, :]

    flat3 = flat.reshape(_NOUT, _N_POINTS // 128, 128)
    out = pl.pallas_call(
        body,
        grid=(_N_POINTS // _RCHUNK,),
        in_specs=[pl.BlockSpec((_NOUT, u, 128), lambda i: (0, i, 0))],
        out_specs=pl.BlockSpec((_NOUT, _RCHUNK), lambda i: (0, i)),
        out_shape=jax.ShapeDtypeStruct((_NOUT, _N_POINTS), jnp.float32),
    )(flat3)
    return out.T


def kernel(x, tables):
    tables_flat = (
        tables.transpose(0, 2, 1)
        .reshape(_N_LEVELS, _NFEAT, _HASHMAP // 128, 128)
        .transpose(0, 2, 1, 3)
        .reshape(-1)
    )
    x_flat = x.reshape(-1)
    res = _resolutions()
    mesh = plsc.VectorSubcoreMesh(core_axis_name="c", subcore_axis_name="s")

    @functools.partial(
        pl.kernel,
        out_type=jax.ShapeDtypeStruct((_N_POINTS * _NOUT,), jnp.float32),
        mesh=mesh,
        compiler_params=pltpu.CompilerParams(needs_layout_passes=False),
        scratch_types=[
            pltpu.VMEM((3 * _C,), jnp.float32),
            pltpu.VMEM((16 * _C,), jnp.int32),
            pltpu.VMEM((16 * _C,), jnp.int32),
            pltpu.VMEM((16 * _C,), jnp.float32),
            pltpu.VMEM((16 * _C,), jnp.float32),
            pltpu.VMEM((3 * _C,), jnp.float32),
            pltpu.VMEM((3 * _C,), jnp.float32),
            pltpu.VMEM((_C * _NOUT,), jnp.float32),
            pltpu.VMEM((_DTOT,), jnp.float32),
            pltpu.SemaphoreType.DMA,
            pltpu.SemaphoreType.DMA,
            pltpu.SemaphoreType.DMA,
            pltpu.SemaphoreType.DMA,
            pltpu.SemaphoreType.DMA,
        ],
    )
    def _k(x_hbm, tab_hbm, out_hbm, xbuf, idxb0, idxb1, featb0, featb1,
           fracb0, fracb1, outbuf, denseb, gsem0a, gsem0b, gsem1a, gsem1b,
           osem):
        wid = lax.axis_index("s") * _NC + lax.axis_index("c")
        base_pt = wid * _NPT
        idxbufs = (idxb0, idxb1)
        featbufs = (featb0, featb1)
        fracbufs = (fracb0, fracb1)
        gsems = ((gsem0a, gsem0b), (gsem1a, gsem1b))
        lanes = lax.iota(jnp.int32, _L)
        lanes3 = lanes * 3

        half = 8 * _C
        for dl in range(_DENSE_L):
            dR = _DENSE_R[dl]
            dplane = _DENSE_PLANE[dl]
            doff = _DENSE_OFF[dl]
            dlvl_base = dl << (_LOG2 + 1)
            n_rows = dR * dR
            rows_per_slice = (half // dR) & ~7
            rows_done = 0
            while rows_done < n_rows:
                nrows_s = min(rows_per_slice, n_rows - rows_done)
                slice_len = nrows_s * dR
                sbase = rows_done * dR
                y0 = rows_done % dR
                z0 = rows_done // dR

                def fill_row(r, yz, dR=dR, dlvl_base=dlvl_base):
                    y, z = yz
                    p0 = r * dR
                    hyz = lax.bitwise_xor(y * _P2, z * _P3)
                    for gx in range(2):
                        xx = gx * _L + lanes
                        xc = jnp.minimum(xx, dR - 1)
                        h = lax.bitwise_and(lax.bitwise_xor(xc, hyz), _MASK)
                        e0 = dlvl_base + h + lax.bitwise_and(h, _BLKMASK)
                        plsc.store_scatter(idxb0, [p0 + xc], e0)
                        plsc.store_scatter(idxb0, [half + p0 + xc], e0 + 128)
                    y1 = y + 1
                    wrap = (y1 == dR).astype(jnp.int32)
                    return (y1 * (1 - wrap), z + wrap)

                lax.fori_loop(0, nrows_s, fill_row,
                              (jnp.int32(y0), jnp.int32(z0)))
                h0 = pltpu.async_copy(
                    tab_hbm.at[idxb0.at[pl.ds(0, slice_len)]],
                    denseb.at[pl.ds(doff + sbase, slice_len)], gsem0a)
                h1 = pltpu.async_copy(
                    tab_hbm.at[idxb0.at[pl.ds(half, slice_len)]],
                    denseb.at[pl.ds(doff + dplane + sbase, slice_len)],
                    gsem0b)
                h0.wait()
                h1.wait()
                rows_done += nrows_s

        def make_pass_a(lvl, idxb, fracb):
            rf = float(res[lvl])
            lvl_base = lvl << (_LOG2 + 1)

            def pass_a(g, carry):
                rows3 = (g * _L) * 3 + lanes3
                xv = plsc.load_gather(xbuf, [rows3])
                yv = plsc.load_gather(xbuf, [rows3 + 1])
                zv = plsc.load_gather(xbuf, [rows3 + 2])
                sx = xv * rf
                sy = yv * rf
                sz = zv * rf
                fxi = sx.astype(jnp.int32)
                fyi = sy.astype(jnp.int32)
                fzi = sz.astype(jnp.int32)
                q = g * _L
                fracb[pl.ds(q, _L)] = sx - fxi.astype(jnp.float32)
                fracb[pl.ds(_C + q, _L)] = sy - fyi.astype(jnp.float32)
                fracb[pl.ds(2 * _C + q, _L)] = sz - fzi.astype(jnp.float32)
                hx0 = fxi
                hx1 = fxi + 1
                hy0 = fyi * _P2
                hy1 = (fyi + 1) * _P2
                hz0 = fzi * _P3
                hz1 = (fzi + 1) * _P3
                corners = ((hx0, hy0, hz0), (hx1, hy0, hz0),
                           (hx0, hy1, hz0), (hx1, hy1, hz0),
                           (hx0, hy0, hz1), (hx1, hy0, hz1),
                           (hx0, hy1, hz1), (hx1, hy1, hz1))
                for c, (hx, hy, hz) in enumerate(corners):
                    h = (hx ^ hy ^ hz) & _MASK
                    e0 = lvl_base + h + (h & _BLKMASK)
                    idxb[pl.ds(c * _C + q, _L)] = e0
                    idxb[pl.ds(8 * _C + c * _C + q, _L)] = e0 + 128
                return carry

            return pass_a

        def make_pass_b(lvl, featb, fracb):
            def pass_b(g, carry):
                q = g * _L
                wx = fracb[pl.ds(q, _L)]
                wy = fracb[pl.ds(_C + q, _L)]
                wz = fracb[pl.ds(2 * _C + q, _L)]
                ux = 1.0 - wx
                uy = 1.0 - wy
                uz = 1.0 - wz
                pa = ux * uy
                pb = wx * uy
                pc = ux * wy
                pd = wx * wy
                ws = (pa * uz, pb * uz, pc * uz, pd * uz,
                      pa * wz, pb * wz, pc * wz, pd * wz)
                acc0 = ws[0] * featb[pl.ds(q, _L)]
                acc1 = ws[0] * featb[pl.ds(8 * _C + q, _L)]
                for c in range(1, 8):
                    acc0 = acc0 + ws[c] * featb[pl.ds(c * _C + q, _L)]
                    acc1 = acc1 + ws[c] * featb[pl.ds(8 * _C + c * _C + q, _L)]
                outbuf[pl.ds(2 * lvl * _C + q, _L)] = acc0
                outbuf[pl.ds((2 * lvl + 1) * _C + q, _L)] = acc1
                return carry

            return pass_b

        def dense_pass(g, carry):
            q = g * _L
            rows3 = q * 3 + lanes3
            xv = plsc.load_gather(xbuf, [rows3])
            yv = plsc.load_gather(xbuf, [rows3 + 1])
            zv = plsc.load_gather(xbuf, [rows3 + 2])
            for dl in range(_DENSE_L):
                dR = _DENSE_R[dl]
                dplane = _DENSE_PLANE[dl]
                doff = _DENSE_OFF[dl]
                rf = float(res[dl])
                sx = xv * rf
                sy = yv * rf
                sz = zv * rf
                fxi = sx.astype(jnp.int32)
                fyi = sy.astype(jnp.int32)
                fzi = sz.astype(jnp.int32)
                wx = sx - fxi.astype(jnp.float32)
                wy = sy - fyi.astype(jnp.float32)
                wz = sz - fzi.astype(jnp.float32)
                base = doff + fxi + (fzi * dR + fyi) * dR
                ux = 1.0 - wx
                uy = 1.0 - wy
                uz = 1.0 - wz
                pa = ux * uy
                pb = wx * uy
                pc = ux * wy
                pd = wx * wy
                ws = (pa * uz, pb * uz, pc * uz, pd * uz,
                      pa * wz, pb * wz, pc * wz, pd * wz)
                kcs = (0, 1, dR, dR + 1,
                       dR * dR, dR * dR + 1, dR * dR + dR, dR * dR + dR + 1)
                ci0 = base + kcs[0]
                acc0 = ws[0] * plsc.load_gather(denseb, [ci0])
                acc1 = ws[0] * plsc.load_gather(denseb, [ci0 + dplane])
                for c in range(1, 8):
                    ci = base + kcs[c]
                    acc0 = acc0 + ws[c] * plsc.load_gather(denseb, [ci])
                    acc1 = acc1 + ws[c] * plsc.load_gather(denseb,
                                                           [ci + dplane])
                outbuf[pl.ds(2 * dl * _C + q, _L)] = acc0
                outbuf[pl.ds((2 * dl + 1) * _C + q, _L)] = acc1
            return carry

        def chunk_body(chunk, carry0):
            c0 = chunk * _C
            pltpu.sync_copy(x_hbm.at[pl.ds((base_pt + c0) * 3, _C * 3)], xbuf)

            ghandles = [None, None]
            ohandles = []

            def emit_out(lvl):
                for t in (0, 1):
                    q = 2 * lvl + t
                    ohandles.append(pltpu.async_copy(
                        outbuf.at[pl.ds(q * _C, _C)],
                        out_hbm.at[pl.ds(q * _N_POINTS + base_pt + c0, _C)],
                        osem,
                    ))

            def start_gather(b):
                idxb, featb, (sa, sb) = idxbufs[b], featbufs[b], gsems[b]
                return (
                    pltpu.async_copy(tab_hbm.at[idxb.at[pl.ds(0, 8 * _C)]],
                                     featb.at[pl.ds(0, 8 * _C)], sa),
                    pltpu.async_copy(
                        tab_hbm.at[idxb.at[pl.ds(8 * _C, 8 * _C)]],
                        featb.at[pl.ds(8 * _C, 8 * _C)], sb),
                )

            first = _DENSE_L
            b0 = first & 1
            lax.fori_loop(0, _G16, make_pass_a(first, idxbufs[b0], fracbufs[b0]), 0)
            ghandles[b0] = start_gather(b0)
            lax.fori_loop(0, _G16, dense_pass, 0)
            for dl in range(_DENSE_L):
                emit_out(dl)
            for lvl in range(first + 1, _N_LEVELS):
                b = lvl & 1
                pb_ = b ^ 1
                lax.fori_loop(0, _G16, make_pass_a(lvl, idxbufs[b], fracbufs[b]), 0)
                ghandles[b] = start_gather(b)
                for h in ghandles[pb_]:
                    h.wait()
                lax.fori_loop(0, _G16, make_pass_b(lvl - 1, featbufs[pb_], fracbufs[pb_]), 0)
                emit_out(lvl - 1)
            last = (_N_LEVELS - 1) & 1
            for h in ghandles[last]:
                h.wait()
            lax.fori_loop(0, _G16, make_pass_b(_N_LEVELS - 1, featbufs[last], fracbufs[last]), 0)
            emit_out(_N_LEVELS - 1)
            for h in ohandles:
                h.wait()
            return carry0

        lax.fori_loop(0, _NCHUNK, chunk_body, 0)

    return _delinearize(_k(x_flat, tables_flat))
